# Initial kernel scaffold; baseline (speedup 1.0000x reference)
import jax
import jax.numpy as jnp
from jax.experimental import pallas as pl


def kernel(partition_indices, queries, states):
    raise NotImplementedError("write your pallas kernel here")


# speedup vs baseline: 1.7378x; 1.7378x over previous
import functools

import jax
import jax.numpy as jnp
from jax import lax
from jax.experimental import pallas as pl
from jax.experimental.pallas import tpu as pltpu
from jax.experimental.pallas import tpu_sc as plsc

M, C, D = 65536, 8, 64
B, S, K = 8, 2048, 4
N = B * S
CD = C * D
L = 16
ND = D // L

NC, NS = 2, 16
NW = NC * NS
TOK_PER_W = N // NW
T = 32
CHUNKS = TOK_PER_W // T


def _sc_read(idx_hbm, q_hbm, st_hbm, out_hbm, idx_v, rows_v, q_v, out_v, sem):
    wid = lax.axis_index("s") * NC + lax.axis_index("c")

    def chunk_body(ch, carry):
        base_tok = wid * TOK_PER_W + ch * T
        pltpu.sync_copy(idx_hbm.at[pl.ds(base_tok * K, T * K)], idx_v)
        cp = pltpu.async_copy(st_hbm.at[idx_v], rows_v, sem)
        pltpu.sync_copy(q_hbm.at[pl.ds(base_tok, T)], q_v)
        cp.wait()

        def tok_body(t, c2):
            for d in range(ND):
                qv = q_v[t, pl.ds(d * L, L)] * (1.0 / C)
                acc = rows_v[t * K, pl.ds(d * L, L)]
                for j in range(1, K * C):
                    k, c = divmod(j, C)
                    acc = acc + rows_v[t * K + k, pl.ds(c * D + d * L, L)]
                out_v[t, pl.ds(d * L, L)] = acc * qv
            return c2

        lax.fori_loop(0, T, tok_body, 0)
        pltpu.sync_copy(out_v, out_hbm.at[pl.ds(base_tok, T)])
        return carry

    lax.fori_loop(0, CHUNKS, chunk_body, 0)


@jax.jit
def _run(idx, q, st):
    f = functools.partial(
        pl.kernel,
        mesh=plsc.VectorSubcoreMesh(core_axis_name="c", subcore_axis_name="s"),
        out_type=jax.ShapeDtypeStruct((N, D), jnp.float32),
        scratch_types=[
            pltpu.VMEM((T * K,), jnp.int32),
            pltpu.VMEM((T * K, CD), jnp.float32),
            pltpu.VMEM((T, D), jnp.float32),
            pltpu.VMEM((T, D), jnp.float32),
            pltpu.SemaphoreType.DMA,
        ],
    )(_sc_read)
    return f(idx, q, st)


def kernel(partition_indices, queries, states):
    idx = partition_indices.reshape(N * K).astype(jnp.int32)
    q = queries.reshape(N, D)
    st = states.reshape(M, CD)
    out = _run(idx, q, st)
    return out.reshape(B, S, D)


# speedup vs baseline: 1.8737x; 1.0782x over previous
import functools

import jax
import jax.numpy as jnp
from jax import lax
from jax.experimental import pallas as pl
from jax.experimental.pallas import tpu as pltpu
from jax.experimental.pallas import tpu_sc as plsc

M, C, D = 65536, 8, 64
B, S, K = 8, 2048, 4
N = B * S
CD = C * D
L = 16
ND = D // L

NC, NS = 2, 16
NW = NC * NS
TOK_PER_W = N // NW
T = 16
CHUNKS = TOK_PER_W // T


def _sc_read(idx_hbm, q_hbm, st_hbm, out_hbm,
             idx0, idx1, rows0, rows1, q0, q1, o0, o1, sem0, sem1):
    wid = lax.axis_index("s") * NC + lax.axis_index("c")
    tok0 = wid * TOK_PER_W
    bufs = ((idx0, rows0, q0, o0, sem0), (idx1, rows1, q1, o1, sem1))

    def issue(ch, b):
        idx_v, rows_v, q_v, _, sem = bufs[b]
        base_tok = tok0 + ch * T
        pltpu.sync_copy(idx_hbm.at[pl.ds(base_tok * K, T * K)], idx_v)
        pltpu.async_copy(st_hbm.at[idx_v], rows_v, sem)
        pltpu.sync_copy(q_hbm.at[pl.ds(base_tok, T)], q_v)

    def finish(ch, b):
        idx_v, rows_v, q_v, out_v, sem = bufs[b]
        pltpu.make_async_copy(st_hbm.at[idx_v], rows_v, sem).wait()

        def tok_body(t, c2):
            for d in range(ND):
                qv = q_v[t, pl.ds(d * L, L)] * (1.0 / C)
                acc = rows_v[t * K, pl.ds(d * L, L)]
                for j in range(1, K * C):
                    k, c = divmod(j, C)
                    acc = acc + rows_v[t * K + k, pl.ds(c * D + d * L, L)]
                out_v[t, pl.ds(d * L, L)] = acc * qv
            return c2

        lax.fori_loop(0, T, tok_body, 0)
        base_tok = tok0 + ch * T
        pltpu.sync_copy(out_v, out_hbm.at[pl.ds(base_tok, T)])

    issue(0, 0)

    def pair_body(i, carry):
        issue(2 * i + 1, 1)
        finish(2 * i, 0)

        @pl.when(i < CHUNKS // 2 - 1)
        def _():
            issue(2 * i + 2, 0)

        finish(2 * i + 1, 1)
        return carry

    lax.fori_loop(0, CHUNKS // 2, pair_body, 0)


@jax.jit
def _run(idx, q, st):
    f = functools.partial(
        pl.kernel,
        mesh=plsc.VectorSubcoreMesh(core_axis_name="c", subcore_axis_name="s"),
        out_type=jax.ShapeDtypeStruct((N, D), jnp.float32),
        scratch_types=[
            pltpu.VMEM((T * K,), jnp.int32),
            pltpu.VMEM((T * K,), jnp.int32),
            pltpu.VMEM((T * K, CD), jnp.float32),
            pltpu.VMEM((T * K, CD), jnp.float32),
            pltpu.VMEM((T, D), jnp.float32),
            pltpu.VMEM((T, D), jnp.float32),
            pltpu.VMEM((T, D), jnp.float32),
            pltpu.VMEM((T, D), jnp.float32),
            pltpu.SemaphoreType.DMA,
            pltpu.SemaphoreType.DMA,
        ],
    )(_sc_read)
    return f(idx, q, st)


def kernel(partition_indices, queries, states):
    idx = partition_indices.reshape(N * K).astype(jnp.int32)
    q = queries.reshape(N, D)
    st = states.reshape(M, CD)
    out = _run(idx, q, st)
    return out.reshape(B, S, D)
